# baseline (speedup 1.0000x reference)
import functools

import jax
import jax.numpy as jnp
from jax import lax
from jax.experimental import pallas as pl
from jax.experimental.pallas import tpu as pltpu
from jax.experimental.pallas import tpu_sc as plsc

N_NODES = 10000
N_EDGES = 320000
D_NODE = 128
D_EDGE = 16
PACK = 128 // D_EDGE
G_ROWS = N_EDGES // PACK

NC = 2
NS = 16
NW = NC * NS

EDGES_PER_W = N_EDGES // NW
MSUB = 100
MNSUB = 4
MCHUNK = MSUB * MNSUB
MPROWS = MCHUNK // PACK
MNCHUNK = EDGES_PER_W // MCHUNK
IDXR_PER_W = EDGES_PER_W // MSUB

RP_EDGES = 320
RP_ROWS = RP_EDGES // PACK
RP_UNITS = N_EDGES // RP_EDGES
RP_ITERS = -(-RP_UNITS // NW)


def _node_proj_kernel(nf_ref, w_ref, p1_ref, p2_ref):
    nf = nf_ref[...]
    w1 = w_ref[0:D_NODE, :]
    w2 = w_ref[D_NODE:2 * D_NODE, :]
    p1_ref[...] = jnp.dot(nf, w1, preferred_element_type=jnp.float32)
    p2_ref[...] = jnp.dot(nf, w2, preferred_element_type=jnp.float32)


def _final_kernel(g_ref, ef_ref, w3_ref, b_ref, out_ref):
    e = jnp.dot(ef_ref[...], w3_ref[...],
                preferred_element_type=jnp.float32) + b_ref[...]
    out_ref[...] = jnp.maximum(g_ref[...] + e, 0.0)


def _sc_edge_kernel(p1_hbm, p2_hbm, idx_hbm, out_hbm,
                    idx_v, g1a, g2a, g1b, g2b, oa, ob,
                    gsa, gsb, osa, osb):
    wid = lax.axis_index("s") * NC + lax.axis_index("c")

    pltpu.sync_copy(idx_hbm.at[0, pl.ds(IDXR_PER_W * wid, IDXR_PER_W)],
                    idx_v.at[0])
    pltpu.sync_copy(idx_hbm.at[1, pl.ds(IDXR_PER_W * wid, IDXR_PER_W)],
                    idx_v.at[1])

    gsets = ((g1a, g2a, gsa, oa, osa), (g1b, g2b, gsb, ob, osb))

    def issue_gathers(k, s):
        g1, g2, gsem = s[0], s[1], s[2]
        for j in range(MNSUB):
            sr = k * MNSUB + j
            pltpu.async_copy(p1_hbm.at[idx_v.at[0, sr]],
                             g1.at[pl.ds(j * MSUB, MSUB)], gsem)
            pltpu.async_copy(p2_hbm.at[idx_v.at[1, sr]],
                             g2.at[pl.ds(j * MSUB, MSUB)], gsem)

    def wait_gathers(s):
        g1, g2, gsem = s[0], s[1], s[2]
        for j in range(MNSUB):
            pltpu.make_async_copy(p1_hbm.at[idx_v.at[0, 0]],
                                  g1.at[pl.ds(j * MSUB, MSUB)], gsem).wait()
            pltpu.make_async_copy(p2_hbm.at[idx_v.at[1, 0]],
                                  g2.at[pl.ds(j * MSUB, MSUB)], gsem).wait()

    def compute(s):
        g1, g2, o = s[0], s[1], s[3]

        def row_body(r):
            for c in range(PACK):
                i = r * PACK + c
                o[r, c * D_EDGE:(c + 1) * D_EDGE] = g1[i, :] + g2[i, :]

        plsc.parallel_loop(0, MPROWS, 1, unroll=2)(row_body)

    def issue_write(k, s):
        o, osem = s[3], s[4]
        row = (wid * MNCHUNK + k) * MPROWS
        pltpu.async_copy(o, out_hbm.at[pl.ds(row, MPROWS)], osem)

    def wait_write(s):
        o, osem = s[3], s[4]
        pltpu.make_async_copy(o, out_hbm.at[pl.ds(0, MPROWS)], osem).wait()

    issue_gathers(0, gsets[0])
    for k in range(MNCHUNK):
        cur = gsets[k % 2]
        if k + 1 < MNCHUNK:
            issue_gathers(k + 1, gsets[(k + 1) % 2])
        wait_gathers(cur)
        if k >= 2:
            wait_write(cur)
        compute(cur)
        issue_write(k, cur)
    wait_write(gsets[(MNCHUNK - 2) % 2])
    wait_write(gsets[(MNCHUNK - 1) % 2])


def _sc_repack_kernel(direction):
    def body(in_hbm, out_hbm, n_a, p_a, n_b, p_b, la, lb, sa, sb):
        wid = lax.axis_index("s") * NC + lax.axis_index("c")

        def uid(i):
            return jnp.minimum(wid * RP_ITERS + i, RP_UNITS - 1)

        sets = ((n_a, p_a, la, sa), (n_b, p_b, lb, sb))

        def issue_load(i, s):
            nb, pb, lsem = s[0], s[1], s[2]
            u = uid(i)
            if direction == "pack":
                pltpu.async_copy(
                    in_hbm.at[pl.ds(u * RP_EDGES, RP_EDGES)], nb, lsem)
            else:
                pltpu.async_copy(
                    in_hbm.at[pl.ds(u * RP_ROWS, RP_ROWS)], pb, lsem)

        def wait_load(s):
            nb, pb, lsem = s[0], s[1], s[2]
            if direction == "pack":
                pltpu.make_async_copy(
                    in_hbm.at[pl.ds(0, RP_EDGES)], nb, lsem).wait()
            else:
                pltpu.make_async_copy(
                    in_hbm.at[pl.ds(0, RP_ROWS)], pb, lsem).wait()

        def mv(s):
            nb, pb = s[0], s[1]
            if direction == "pack":
                def f(r):
                    for c in range(PACK):
                        pb[r, c * D_EDGE:(c + 1) * D_EDGE] = (
                            nb[r * PACK + c, :])
            else:
                def f(r):
                    for c in range(PACK):
                        nb[r * PACK + c, :] = (
                            pb[r, c * D_EDGE:(c + 1) * D_EDGE])
            plsc.parallel_loop(0, RP_ROWS, 1, unroll=2)(f)

        def issue_store(i, s):
            nb, pb, ssem = s[0], s[1], s[3]
            u = uid(i)
            if direction == "pack":
                pltpu.async_copy(
                    pb, out_hbm.at[pl.ds(u * RP_ROWS, RP_ROWS)], ssem)
            else:
                pltpu.async_copy(
                    nb, out_hbm.at[pl.ds(u * RP_EDGES, RP_EDGES)], ssem)

        def wait_store(s):
            nb, pb, ssem = s[0], s[1], s[3]
            if direction == "pack":
                pltpu.make_async_copy(
                    pb, out_hbm.at[pl.ds(0, RP_ROWS)], ssem).wait()
            else:
                pltpu.make_async_copy(
                    nb, out_hbm.at[pl.ds(0, RP_EDGES)], ssem).wait()

        issue_load(0, sets[0])
        for k in range(RP_ITERS):
            cur = sets[k % 2]
            if k + 1 < RP_ITERS:
                issue_load(k + 1, sets[(k + 1) % 2])
            wait_load(cur)
            if k >= 2:
                wait_store(cur)
            mv(cur)
            issue_store(k, cur)
        wait_store(sets[(RP_ITERS - 2) % 2])
        wait_store(sets[(RP_ITERS - 1) % 2])

    return body


def kernel(node_feat, edge_index, edge_feat, W, b):
    p1, p2 = pl.pallas_call(
        _node_proj_kernel,
        grid=(10,),
        in_specs=[
            pl.BlockSpec((N_NODES // 10, D_NODE), lambda i: (i, 0)),
            pl.BlockSpec((2 * D_NODE, D_EDGE), lambda i: (0, 0)),
        ],
        out_specs=[
            pl.BlockSpec((N_NODES // 10, D_EDGE), lambda i: (i, 0)),
            pl.BlockSpec((N_NODES // 10, D_EDGE), lambda i: (i, 0)),
        ],
        out_shape=[
            jax.ShapeDtypeStruct((N_NODES, D_EDGE), jnp.float32),
            jax.ShapeDtypeStruct((N_NODES, D_EDGE), jnp.float32),
        ],
    )(node_feat, W[:2 * D_NODE])

    mesh = plsc.VectorSubcoreMesh(
        core_axis_name="c", subcore_axis_name="s",
        num_cores=NC, num_subcores=NS)
    sc_linear = pltpu.CompilerParams(use_tc_tiling_on_sc=False)
    sc_tiled = pltpu.CompilerParams(use_tc_tiling_on_sc=True)
    repack_scratch = [
        pltpu.VMEM((RP_EDGES, D_EDGE), jnp.float32),
        pltpu.VMEM((RP_ROWS, PACK * D_EDGE), jnp.float32),
        pltpu.VMEM((RP_EDGES, D_EDGE), jnp.float32),
        pltpu.VMEM((RP_ROWS, PACK * D_EDGE), jnp.float32),
        pltpu.SemaphoreType.DMA,
        pltpu.SemaphoreType.DMA,
        pltpu.SemaphoreType.DMA,
        pltpu.SemaphoreType.DMA,
    ]

    idx3d = edge_index.astype(jnp.int32).reshape(2, N_EDGES // MSUB, MSUB)
    g_packed = functools.partial(
        pl.kernel,
        out_type=jax.ShapeDtypeStruct((G_ROWS, PACK * D_EDGE), jnp.float32),
        mesh=mesh,
        scratch_types=[
            pltpu.VMEM((2, IDXR_PER_W, MSUB), jnp.int32),
            pltpu.VMEM((MCHUNK, D_EDGE), jnp.float32),
            pltpu.VMEM((MCHUNK, D_EDGE), jnp.float32),
            pltpu.VMEM((MCHUNK, D_EDGE), jnp.float32),
            pltpu.VMEM((MCHUNK, D_EDGE), jnp.float32),
            pltpu.VMEM((MPROWS, PACK * D_EDGE), jnp.float32),
            pltpu.VMEM((MPROWS, PACK * D_EDGE), jnp.float32),
            pltpu.SemaphoreType.DMA,
            pltpu.SemaphoreType.DMA,
            pltpu.SemaphoreType.DMA,
            pltpu.SemaphoreType.DMA,
        ],
        compiler_params=sc_linear,
    )(_sc_edge_kernel)(p1, p2, idx3d)

    ef_cg = functools.partial(
        pl.kernel,
        out_type=jax.ShapeDtypeStruct((G_ROWS, PACK * D_EDGE), jnp.float32),
        mesh=mesh,
        scratch_types=repack_scratch,
        compiler_params=sc_tiled,
    )(_sc_repack_kernel("pack"))(edge_feat)

    NBLK = 40
    w3_big = jnp.kron(jnp.eye(PACK, dtype=jnp.float32), W[2 * D_NODE:])
    b_big = jnp.tile(b, PACK).reshape(1, PACK * D_EDGE)
    out_packed = pl.pallas_call(
        _final_kernel,
        grid=(NBLK,),
        in_specs=[
            pl.BlockSpec((G_ROWS // NBLK, PACK * D_EDGE), lambda i: (i, 0)),
            pl.BlockSpec((G_ROWS // NBLK, PACK * D_EDGE), lambda i: (i, 0)),
            pl.BlockSpec((PACK * D_EDGE, PACK * D_EDGE), lambda i: (0, 0)),
            pl.BlockSpec((1, PACK * D_EDGE), lambda i: (0, 0)),
        ],
        out_specs=pl.BlockSpec((G_ROWS // NBLK, PACK * D_EDGE), lambda i: (i, 0)),
        out_shape=jax.ShapeDtypeStruct((G_ROWS, PACK * D_EDGE), jnp.float32),
    )(g_packed, ef_cg, w3_big, b_big)

    return functools.partial(
        pl.kernel,
        out_type=jax.ShapeDtypeStruct((N_EDGES, D_EDGE), jnp.float32),
        mesh=mesh,
        scratch_types=repack_scratch,
        compiler_params=sc_tiled,
    )(_sc_repack_kernel("unpack"))(out_packed)


# speedup vs baseline: 2.4236x; 2.4236x over previous
import functools

import jax
import jax.numpy as jnp
from jax import lax
from jax.experimental import pallas as pl
from jax.experimental.pallas import tpu as pltpu
from jax.experimental.pallas import tpu_sc as plsc

N_NODES = 10000
N_EDGES = 320000
D_NODE = 128
D_EDGE = 16

NC = 2
NS = 16
NW = NC * NS

USUB = 128
UNSUB = 4
UEDGES = USUB * UNSUB
UNITS = N_EDGES // UEDGES
UITERS = -(-UNITS // NW)
IDXROWS = N_EDGES // USUB


def _node_proj_kernel(nf_ref, w_ref, p1_ref, p2_ref):
    nf = nf_ref[...]
    w1 = w_ref[0:D_NODE, :]
    w2 = w_ref[D_NODE:2 * D_NODE, :]
    p1_ref[...] = jnp.dot(nf, w1, preferred_element_type=jnp.float32)
    p2_ref[...] = jnp.dot(nf, w2, preferred_element_type=jnp.float32)


def _final_kernel(g_ref, ef_ref, w3_ref, b_ref, out_ref):
    e = lax.dot_general(w3_ref[...], ef_ref[...],
                        (((0,), (0,)), ((), ())),
                        preferred_element_type=jnp.float32)
    bb = jnp.broadcast_to(b_ref[...], e.shape)
    out_ref[...] = jnp.maximum(g_ref[...] + e + bb, 0.0)


def _sc_edge_kernel(p1_hbm, p2_hbm, idx_hbm, out_hbm,
                    sa, da, sb, db, g1a, g2a, g1b, g2b, oa, ob,
                    isa_, isb_, gsa, gsb, osa, osb):
    wid = lax.axis_index("s") * NC + lax.axis_index("c")
    lane = lax.iota(jnp.int32, D_EDGE)

    sets = ((sa, da, g1a, g2a, oa, isa_, gsa, osa),
            (sb, db, g1b, g2b, ob, isb_, gsb, osb))

    def uid(i):
        return jnp.minimum(wid * UITERS + i, UNITS - 1)

    def issue_idx(i, s):
        sv, dv, isem = s[0], s[1], s[5]
        u = uid(i)
        pltpu.async_copy(idx_hbm.at[0, pl.ds(u * UNSUB, UNSUB)], sv, isem)
        pltpu.async_copy(idx_hbm.at[1, pl.ds(u * UNSUB, UNSUB)], dv, isem)

    def wait_idx(s):
        sv, dv, isem = s[0], s[1], s[5]
        pltpu.make_async_copy(idx_hbm.at[0, pl.ds(0, UNSUB)], sv, isem).wait()
        pltpu.make_async_copy(idx_hbm.at[1, pl.ds(0, UNSUB)], dv, isem).wait()

    def issue_gathers(s):
        sv, dv, g1, g2, gsem = s[0], s[1], s[2], s[3], s[6]
        for j in range(UNSUB):
            pltpu.async_copy(p1_hbm.at[sv.at[j]],
                             g1.at[pl.ds(j * USUB, USUB)], gsem)
            pltpu.async_copy(p2_hbm.at[dv.at[j]],
                             g2.at[pl.ds(j * USUB, USUB)], gsem)

    def wait_gathers(s):
        sv, g1, g2, gsem = s[0], s[2], s[3], s[6]
        for j in range(UNSUB):
            pltpu.make_async_copy(p1_hbm.at[sv.at[0]],
                                  g1.at[pl.ds(j * USUB, USUB)], gsem).wait()
            pltpu.make_async_copy(p1_hbm.at[sv.at[0]],
                                  g2.at[pl.ds(j * USUB, USUB)], gsem).wait()

    def compute(s):
        g1, g2, o = s[2], s[3], s[4]
        zeros = lane * 0

        def row_body(i):
            v = g1[i, :] + g2[i, :]
            plsc.store_scatter(o, [lane, zeros + i], v)

        plsc.parallel_loop(0, UEDGES, 1, unroll=4)(row_body)

    def issue_write(i, s):
        o, osem = s[4], s[7]
        u = uid(i)
        pltpu.async_copy(o, out_hbm.at[:, pl.ds(u * UEDGES, UEDGES)], osem)

    def wait_write(s):
        o, osem = s[4], s[7]
        pltpu.make_async_copy(
            o, out_hbm.at[:, pl.ds(0, UEDGES)], osem).wait()

    issue_idx(0, sets[0])
    wait_idx(sets[0])
    issue_gathers(sets[0])
    issue_idx(1, sets[1])
    for k in range(UITERS):
        cur = sets[k % 2]
        nxt = sets[(k + 1) % 2]
        if k + 1 < UITERS:
            wait_idx(nxt)
            issue_gathers(nxt)
        wait_gathers(cur)
        if k + 2 < UITERS:
            issue_idx(k + 2, cur)
        if k >= 2:
            wait_write(cur)
        compute(cur)
        issue_write(k, cur)
    wait_write(sets[(UITERS - 2) % 2])
    wait_write(sets[(UITERS - 1) % 2])


def kernel(node_feat, edge_index, edge_feat, W, b):
    p1, p2 = pl.pallas_call(
        _node_proj_kernel,
        grid=(10,),
        in_specs=[
            pl.BlockSpec((N_NODES // 10, D_NODE), lambda i: (i, 0)),
            pl.BlockSpec((2 * D_NODE, D_EDGE), lambda i: (0, 0)),
        ],
        out_specs=[
            pl.BlockSpec((N_NODES // 10, D_EDGE), lambda i: (i, 0)),
            pl.BlockSpec((N_NODES // 10, D_EDGE), lambda i: (i, 0)),
        ],
        out_shape=[
            jax.ShapeDtypeStruct((N_NODES, D_EDGE), jnp.float32),
            jax.ShapeDtypeStruct((N_NODES, D_EDGE), jnp.float32),
        ],
    )(node_feat, W[:2 * D_NODE])

    idx3d = edge_index.astype(jnp.int32).reshape(2, IDXROWS, USUB)
    mesh = plsc.VectorSubcoreMesh(
        core_axis_name="c", subcore_axis_name="s",
        num_cores=NC, num_subcores=NS)
    g3 = functools.partial(
        pl.kernel,
        out_type=jax.ShapeDtypeStruct((D_EDGE, N_EDGES), jnp.float32),
        mesh=mesh,
        scratch_types=[
            pltpu.VMEM((UNSUB, USUB), jnp.int32),
            pltpu.VMEM((UNSUB, USUB), jnp.int32),
            pltpu.VMEM((UNSUB, USUB), jnp.int32),
            pltpu.VMEM((UNSUB, USUB), jnp.int32),
            pltpu.VMEM((UEDGES, D_EDGE), jnp.float32),
            pltpu.VMEM((UEDGES, D_EDGE), jnp.float32),
            pltpu.VMEM((UEDGES, D_EDGE), jnp.float32),
            pltpu.VMEM((UEDGES, D_EDGE), jnp.float32),
            pltpu.VMEM((D_EDGE, UEDGES), jnp.float32),
            pltpu.VMEM((D_EDGE, UEDGES), jnp.float32),
            pltpu.SemaphoreType.DMA,
            pltpu.SemaphoreType.DMA,
            pltpu.SemaphoreType.DMA,
            pltpu.SemaphoreType.DMA,
            pltpu.SemaphoreType.DMA,
            pltpu.SemaphoreType.DMA,
        ],
        compiler_params=pltpu.CompilerParams(
            use_tc_tiling_on_sc=False, needs_layout_passes=False),
    )(_sc_edge_kernel)(p1, p2, idx3d)

    g_t = g3
    ef_t = edge_feat.T
    NBLK = 25
    BLK = N_EDGES // NBLK
    out_t = pl.pallas_call(
        _final_kernel,
        grid=(NBLK,),
        in_specs=[
            pl.BlockSpec((D_EDGE, BLK), lambda i: (0, i)),
            pl.BlockSpec((D_EDGE, BLK), lambda i: (0, i)),
            pl.BlockSpec((D_EDGE, D_EDGE), lambda i: (0, 0)),
            pl.BlockSpec((D_EDGE, 1), lambda i: (0, 0)),
        ],
        out_specs=pl.BlockSpec((D_EDGE, BLK), lambda i: (0, i)),
        out_shape=jax.ShapeDtypeStruct((D_EDGE, N_EDGES), jnp.float32),
    )(g_t, ef_t, W[2 * D_NODE:], b.reshape(D_EDGE, 1))
    return out_t.T
